# baseline (speedup 1.0000x reference)
import functools

import jax
import jax.numpy as jnp
from jax import lax
from jax.experimental import pallas as pl
from jax.experimental.pallas import tpu as pltpu
from jax.experimental.pallas import tpu_sc as plsc

N = 10000
E = 320000
D = 128
ROWS = 2000

NC = 2
NS = 16
L = 16
NW = NC * NS
EP = E // NW
K = 80
NCHUNK = EP // K
NG = K // L
RCH = 16
NCH = N // RCH
DCH = 512
NDC = N // DCH


def _lrelu(z):
    return jnp.where(z > 0, z, 0.2 * z)


def _proj_body(x_ref, wl_ref, bl_ref, wr_ref, br_ref, wres_ref,
               xl_ref, xr_ref, res_ref):
    xb = x_ref[...]
    xl_ref[...] = jnp.dot(xb, wl_ref[...],
                          preferred_element_type=jnp.float32) + bl_ref[...][None, :]
    xr_ref[...] = jnp.dot(xb, wr_ref[...],
                          preferred_element_type=jnp.float32) + br_ref[...][None, :]
    res_ref[...] = jnp.dot(xb, wres_ref[...], preferred_element_type=jnp.float32)


def _project(x, W_l, b_l, W_r, b_r, W_res):
    grid = (N // ROWS,)
    rb = pl.BlockSpec((ROWS, D), lambda i: (i, 0))
    full = pl.BlockSpec((D, D), lambda i: (0, 0))
    vec = pl.BlockSpec((D,), lambda i: (0,))
    return pl.pallas_call(
        _proj_body,
        grid=grid,
        in_specs=[rb, full, vec, full, vec, full],
        out_specs=[rb, rb, rb],
        out_shape=[jax.ShapeDtypeStruct((N, D), jnp.float32)] * 3,
    )(x, W_l, b_l, W_r, b_r, W_res)


def _post_h(xl, xr, res, o_ref, d0, d1, att, bias, g, be):
    lr = _lrelu(xl + xr)
    aii = jnp.sum(lr * att, axis=-1, keepdims=True)
    exii = jnp.exp(aii)
    num = o_ref[0] + o_ref[1] + exii * xl
    den = d0 + d1 + exii
    out = num / (den + 1e-16)
    out = out + res + bias[None, :]
    mu = jnp.mean(out, axis=-1, keepdims=True)
    var = jnp.mean((out - mu) ** 2, axis=-1, keepdims=True)
    out = (out - mu) * lax.rsqrt(var + 1e-5)
    out = out * g[None, :] + be[None, :]
    return jnp.maximum(out, 0.0)


def _post_body(xl_ref, xr_ref, res_ref, o_ref, d0_ref, d1_ref,
               att_ref, bias_ref, g_ref, be_ref, out_ref):
    out_ref[...] = _post_h(xl_ref[...], xr_ref[...], res_ref[...], o_ref,
                           d0_ref[...], d1_ref[...], att_ref[...],
                           bias_ref[...], g_ref[...], be_ref[...])


def _mid_body(xl_ref, xr_ref, res_ref, o_ref, d0_ref, d1_ref,
              att_ref, bias_ref, g_ref, be_ref,
              wl_ref, bl_ref, wr_ref, br_ref, wres_ref,
              xl1_ref, xr1_ref, res1_ref):
    h = _post_h(xl_ref[...], xr_ref[...], res_ref[...], o_ref,
                d0_ref[...], d1_ref[...], att_ref[...],
                bias_ref[...], g_ref[...], be_ref[...])
    xl1_ref[...] = jnp.dot(h, wl_ref[...],
                           preferred_element_type=jnp.float32) + bl_ref[...][None, :]
    xr1_ref[...] = jnp.dot(h, wr_ref[...],
                           preferred_element_type=jnp.float32) + br_ref[...][None, :]
    res1_ref[...] = jnp.dot(h, wres_ref[...], preferred_element_type=jnp.float32)


_rb = pl.BlockSpec((ROWS, D), lambda i: (i, 0))
_cb = pl.BlockSpec((ROWS, 1), lambda i: (i, 0))
_ab = pl.BlockSpec((1, D), lambda i: (0, 0))
_ob = pl.BlockSpec((2, ROWS, D), lambda i: (0, i, 0))
_full = pl.BlockSpec((D, D), lambda i: (0, 0))
_vec = pl.BlockSpec((D,), lambda i: (0,))


def _post(xl, xr, res, o, d0, d1, att, bias, g, be):
    return pl.pallas_call(
        _post_body,
        grid=(N // ROWS,),
        in_specs=[_rb, _rb, _rb, _ob, _cb, _cb, _ab, _vec, _vec, _vec],
        out_specs=_rb,
        out_shape=jax.ShapeDtypeStruct((N, D), jnp.float32),
    )(xl, xr, res, o, d0, d1, att, bias, g, be)


def _mid(xl, xr, res, o, d0, d1, att, bias, g, be, W_l, b_l, W_r, b_r, W_res):
    return pl.pallas_call(
        _mid_body,
        grid=(N // ROWS,),
        in_specs=[_rb, _rb, _rb, _ob, _cb, _cb, _ab, _vec, _vec, _vec,
                  _full, _vec, _full, _vec, _full],
        out_specs=[_rb, _rb, _rb],
        out_shape=[jax.ShapeDtypeStruct((N, D), jnp.float32)] * 3,
    )(xl, xr, res, o, d0, d1, att, bias, g, be, W_l, b_l, W_r, b_r, W_res)


def _edge_body(xl_hbm, xr_hbm, att_hbm, src_hbm, dst_hbm,
               out_hbm, den0_hbm, den1_hbm,
               src_i0, dst_i0, src_i1, dst_i1, dsc_i0, dsc_i1,
               S0, T0, S1, T1, EX0, EX1, attv, Z, Zd,
               out_acc, den_acc, gsem0, gsem1, isem0, isem1, ssem0, ssem1):
    cid = lax.axis_index("c")
    sid = lax.axis_index("s")
    wid = sid * NC + cid
    src_i = (src_i0, src_i1)
    dst_i = (dst_i0, dst_i1)
    dsc_i = (dsc_i0, dsc_i1)
    S = (S0, S1)
    T = (T0, T1)
    EX = (EX0, EX1)
    gsem = (gsem0, gsem1)
    isem = (isem0, isem1)
    ssem = (ssem0, ssem1)

    zv = jnp.zeros((L,), jnp.float32)

    def _zrow(i, _):
        for j in range(8):
            Z[i, pl.ds(16 * j, 16)] = zv
        return 0
    lax.fori_loop(0, RCH, _zrow, 0)

    def _zd(i, _):
        Zd[pl.ds(i * 16, 16)] = zv
        return 0
    lax.fori_loop(0, DCH // 16, _zd, 0)

    def _zacc(k, _):
        ch = sid + 16 * k

        @pl.when(ch < NCH)
        def _():
            pltpu.sync_copy(Z, out_acc.at[pl.ds(ch * RCH, RCH)])
        return 0

    lax.fori_loop(0, (NCH + 15) // 16, _zacc, 0)

    for k in range(2):
        ch = sid + 16 * k

        @pl.when(ch < NDC)
        def _():
            pltpu.sync_copy(Zd, den_acc.at[pl.ds(ch * DCH, DCH)])

        @pl.when(ch == NDC)
        def _():
            pltpu.sync_copy(Zd.at[pl.ds(0, N - NDC * DCH)],
                            den_acc.at[pl.ds(NDC * DCH, N - NDC * DCH)])

    plsc.subcore_barrier()

    pltpu.sync_copy(att_hbm, attv)
    att_regs = [attv[pl.ds(16 * j, 16)] for j in range(8)]
    rows16 = jnp.arange(16, dtype=jnp.int32)
    lane0 = rows16 == 0
    ebase = wid * EP

    def _fetch_idx(c, b):
        pltpu.async_copy(src_hbm.at[pl.ds(ebase + c * K, K)], src_i[b], isem[b])
        pltpu.async_copy(dst_hbm.at[pl.ds(ebase + c * K, K)], dst_i[b], isem[b])

    def _wait_idx(b):
        pltpu.make_async_copy(src_hbm.at[pl.ds(ebase, K)], src_i[b],
                              isem[b]).wait()
        pltpu.make_async_copy(dst_hbm.at[pl.ds(ebase, K)], dst_i[b],
                              isem[b]).wait()

    def _gather(b):
        pltpu.async_copy(xl_hbm.at[src_i[b]], S[b], gsem[b])
        pltpu.async_copy(xr_hbm.at[dst_i[b]], T[b], gsem[b])

    def _wait_gather(b):
        pltpu.make_async_copy(xl_hbm.at[src_i[b]], S[b], gsem[b]).wait()
        pltpu.make_async_copy(xr_hbm.at[dst_i[b]], T[b], gsem[b]).wait()

    def _scatter(b):
        pltpu.async_copy(S[b], out_acc.at[dsc_i[b]], ssem[b], add=True)
        pltpu.async_copy(EX[b], den_acc.at[dsc_i[b]], ssem[b], add=True)

    def _wait_scatter(b):
        pltpu.make_async_copy(S[b], out_acc.at[dsc_i[b]], ssem[b]).wait()
        pltpu.make_async_copy(EX[b], den_acc.at[dsc_i[b]], ssem[b]).wait()

    def _compute(b):
        Sb, Tb = S[b], T[b]

        def _cpi(g, _):
            dsc_i[b][pl.ds(g * L, L)] = dst_i[b][pl.ds(g * L, L)]
            return 0

        lax.fori_loop(0, NG, _cpi, 0)

        @plsc.parallel_loop(0, K, unroll=2)
        def _edge(ee):
            eidx = jnp.full((L,), ee, jnp.int32)
            srcs = plsc.load_gather(src_i[b], [eidx])
            dsts = plsc.load_gather(dst_i[b], [eidx])
            srow = [Sb[ee, pl.ds(16 * j, 16)] for j in range(8)]
            acc = zv
            for j in range(8):
                z = srow[j] + Tb[ee, pl.ds(16 * j, 16)]
                acc = acc + jnp.maximum(z, 0.2 * z) * att_regs[j]
            al = jnp.sum(acc)
            exs = jnp.where(srcs != dsts, jnp.exp(jnp.full((L,), al)), 0.0)
            for j in range(8):
                Sb[ee, pl.ds(16 * j, 16)] = srow[j] * exs
            plsc.store_scatter(EX[b], [eidx], exs, mask=lane0)

    pltpu.sync_copy(src_hbm.at[pl.ds(ebase, K)], src_i[0])
    pltpu.sync_copy(dst_hbm.at[pl.ds(ebase, K)], dst_i[0])
    _gather(0)
    _fetch_idx(1, 1)

    def _pair(i, _):
        for b in range(2):
            c = 2 * i + b

            @pl.when(c + 1 < NCHUNK)
            def _():
                _wait_idx(b ^ 1)

                @pl.when(c >= 1)
                def _():
                    _wait_scatter(b ^ 1)

                _gather(b ^ 1)

            _wait_gather(b)
            _compute(b)

            @pl.when(c + 2 < NCHUNK)
            def _():
                _fetch_idx(c + 2, b)

            _scatter(b)
        return 0

    lax.fori_loop(0, NCHUNK // 2, _pair, 0)
    _wait_gather(0)
    _compute(0)
    _wait_scatter(1)
    _scatter(0)
    _wait_scatter(0)

    plsc.subcore_barrier()

    def _dracc(k, _):
        ch = sid + 16 * k

        @pl.when(ch < NCH)
        def _():
            pltpu.sync_copy(out_acc.at[pl.ds(ch * RCH, RCH)], Z)
            pltpu.sync_copy(Z, out_hbm.at[cid, pl.ds(ch * RCH, RCH)])
        return 0

    lax.fori_loop(0, (NCH + 15) // 16, _dracc, 0)

    def _dr_den(den_dst):
        for k in range(2):
            ch = sid + 16 * k

            @pl.when(ch < NDC)
            def _():
                pltpu.sync_copy(den_acc.at[pl.ds(ch * DCH, DCH)], Zd)
                pltpu.sync_copy(Zd, den_dst.at[pl.ds(ch * DCH, DCH)])

            @pl.when(ch == NDC)
            def _():
                nrem = N - NDC * DCH
                pltpu.sync_copy(den_acc.at[pl.ds(NDC * DCH, nrem)],
                                Zd.at[pl.ds(0, nrem)])
                pltpu.sync_copy(Zd.at[pl.ds(0, nrem)],
                                den_dst.at[pl.ds(NDC * DCH, nrem)])

    @pl.when(cid == 0)
    def _():
        _dr_den(den0_hbm)

    @pl.when(cid == 1)
    def _():
        _dr_den(den1_hbm)


@functools.partial(jax.jit, static_argnums=())
def _edges(xl, xr, src, dst, att):
    f = pl.kernel(
        _edge_body,
        mesh=plsc.VectorSubcoreMesh(core_axis_name="c", subcore_axis_name="s",
                                    num_cores=NC),
        compiler_params=pltpu.CompilerParams(needs_layout_passes=False),
        out_type=[
            jax.ShapeDtypeStruct((NC, N, D), jnp.float32),
            jax.ShapeDtypeStruct((N,), jnp.float32),
            jax.ShapeDtypeStruct((N,), jnp.float32),
        ],
        scratch_types=[
            pltpu.VMEM((K,), jnp.int32),
            pltpu.VMEM((K,), jnp.int32),
            pltpu.VMEM((K,), jnp.int32),
            pltpu.VMEM((K,), jnp.int32),
            pltpu.VMEM((K,), jnp.int32),
            pltpu.VMEM((K,), jnp.int32),
            pltpu.VMEM((K, D), jnp.float32),
            pltpu.VMEM((K, D), jnp.float32),
            pltpu.VMEM((K, D), jnp.float32),
            pltpu.VMEM((K, D), jnp.float32),
            pltpu.VMEM((K,), jnp.float32),
            pltpu.VMEM((K,), jnp.float32),
            pltpu.VMEM((D,), jnp.float32),
            pltpu.VMEM((RCH, D), jnp.float32),
            pltpu.VMEM((DCH,), jnp.float32),
            pltpu.VMEM_SHARED((N, D), jnp.float32),
            pltpu.VMEM_SHARED((N,), jnp.float32),
            pltpu.SemaphoreType.DMA,
            pltpu.SemaphoreType.DMA,
            pltpu.SemaphoreType.DMA,
            pltpu.SemaphoreType.DMA,
            pltpu.SemaphoreType.DMA,
            pltpu.SemaphoreType.DMA,
        ],
    )
    return f(xl, xr, att.reshape(D), src, dst)


def kernel(x, edge_index, W_l0, b_l0, W_r0, b_r0, att0, bias0, W_res0, g0, be0,
           W_l1, b_l1, W_r1, b_r1, att1, bias1, W_res1, g1, be1):
    src = edge_index[0]
    dst = edge_index[1]
    xl0, xr0, res0 = _project(x, W_l0, b_l0, W_r0, b_r0, W_res0)
    o, d0, d1 = _edges(xl0, xr0, src, dst, att0)
    xl1, xr1, res1 = _mid(xl0, xr0, res0, o, d0.reshape(N, 1),
                          d1.reshape(N, 1), att0, bias0, g0, be0,
                          W_l1, b_l1, W_r1, b_r1, W_res1)
    o, d0, d1 = _edges(xl1, xr1, src, dst, att1)
    return _post(xl1, xr1, res1, o, d0.reshape(N, 1), d1.reshape(N, 1),
                 att1, bias1, g1, be1)


# speedup vs baseline: 1.1434x; 1.1434x over previous
import functools

import jax
import jax.numpy as jnp
from jax import lax
from jax.experimental import pallas as pl
from jax.experimental.pallas import tpu as pltpu
from jax.experimental.pallas import tpu_sc as plsc

N = 10000
E = 320000
D = 128
ROWS = 2000

NC = 2
NS = 16
L = 16
NW = NC * NS
EP = E // NW
K = 80
NCHUNK = EP // K
NG = K // L
RCH = 16
NCH = N // RCH
DCH = 512
NDC = N // DCH


def _lrelu(z):
    return jnp.where(z > 0, z, 0.2 * z)


def _proj_body(x_ref, wl_ref, bl_ref, wr_ref, br_ref, wres_ref,
               xl_ref, xr_ref, res_ref):
    xb = x_ref[...]
    xl_ref[...] = jnp.dot(xb, wl_ref[...],
                          preferred_element_type=jnp.float32) + bl_ref[...][None, :]
    xr_ref[...] = jnp.dot(xb, wr_ref[...],
                          preferred_element_type=jnp.float32) + br_ref[...][None, :]
    res_ref[...] = jnp.dot(xb, wres_ref[...], preferred_element_type=jnp.float32)


def _project(x, W_l, b_l, W_r, b_r, W_res):
    grid = (N // ROWS,)
    rb = pl.BlockSpec((ROWS, D), lambda i: (i, 0))
    full = pl.BlockSpec((D, D), lambda i: (0, 0))
    vec = pl.BlockSpec((D,), lambda i: (0,))
    return pl.pallas_call(
        _proj_body,
        grid=grid,
        in_specs=[rb, full, vec, full, vec, full],
        out_specs=[rb, rb, rb],
        out_shape=[jax.ShapeDtypeStruct((N, D), jnp.float32)] * 3,
    )(x, W_l, b_l, W_r, b_r, W_res)


def _post_h(xl, xr, res, o_ref, d0, d1, att, bias, g, be):
    lr = _lrelu(xl + xr)
    aii = jnp.sum(lr * att, axis=-1, keepdims=True)
    exii = jnp.exp(aii)
    num = o_ref[0] + o_ref[1] + exii * xl
    den = d0 + d1 + exii
    out = num / (den + 1e-16)
    out = out + res + bias[None, :]
    mu = jnp.mean(out, axis=-1, keepdims=True)
    var = jnp.mean((out - mu) ** 2, axis=-1, keepdims=True)
    out = (out - mu) * lax.rsqrt(var + 1e-5)
    out = out * g[None, :] + be[None, :]
    return jnp.maximum(out, 0.0)


def _post_body(xl_ref, xr_ref, res_ref, o_ref, d0_ref, d1_ref,
               att_ref, bias_ref, g_ref, be_ref, out_ref):
    out_ref[...] = _post_h(xl_ref[...], xr_ref[...], res_ref[...], o_ref,
                           d0_ref[...], d1_ref[...], att_ref[...],
                           bias_ref[...], g_ref[...], be_ref[...])


def _mid_body(xl_ref, xr_ref, res_ref, o_ref, d0_ref, d1_ref,
              att_ref, bias_ref, g_ref, be_ref,
              wl_ref, bl_ref, wr_ref, br_ref, wres_ref,
              xl1_ref, xr1_ref, res1_ref):
    h = _post_h(xl_ref[...], xr_ref[...], res_ref[...], o_ref,
                d0_ref[...], d1_ref[...], att_ref[...],
                bias_ref[...], g_ref[...], be_ref[...])
    xl1_ref[...] = jnp.dot(h, wl_ref[...],
                           preferred_element_type=jnp.float32) + bl_ref[...][None, :]
    xr1_ref[...] = jnp.dot(h, wr_ref[...],
                           preferred_element_type=jnp.float32) + br_ref[...][None, :]
    res1_ref[...] = jnp.dot(h, wres_ref[...], preferred_element_type=jnp.float32)


_rb = pl.BlockSpec((ROWS, D), lambda i: (i, 0))
_cb = pl.BlockSpec((ROWS, 1), lambda i: (i, 0))
_ab = pl.BlockSpec((1, D), lambda i: (0, 0))
_ob = pl.BlockSpec((2, ROWS, D), lambda i: (0, i, 0))
_full = pl.BlockSpec((D, D), lambda i: (0, 0))
_vec = pl.BlockSpec((D,), lambda i: (0,))


def _post(xl, xr, res, o, d0, d1, att, bias, g, be):
    return pl.pallas_call(
        _post_body,
        grid=(N // ROWS,),
        in_specs=[_rb, _rb, _rb, _ob, _cb, _cb, _ab, _vec, _vec, _vec],
        out_specs=_rb,
        out_shape=jax.ShapeDtypeStruct((N, D), jnp.float32),
    )(xl, xr, res, o, d0, d1, att, bias, g, be)


def _mid(xl, xr, res, o, d0, d1, att, bias, g, be, W_l, b_l, W_r, b_r, W_res):
    return pl.pallas_call(
        _mid_body,
        grid=(N // ROWS,),
        in_specs=[_rb, _rb, _rb, _ob, _cb, _cb, _ab, _vec, _vec, _vec,
                  _full, _vec, _full, _vec, _full],
        out_specs=[_rb, _rb, _rb],
        out_shape=[jax.ShapeDtypeStruct((N, D), jnp.float32)] * 3,
    )(xl, xr, res, o, d0, d1, att, bias, g, be, W_l, b_l, W_r, b_r, W_res)


def _edge_body(xl_hbm, xr_hbm, att_hbm, src_hbm, dst_hbm,
               out_hbm, den0_hbm, den1_hbm,
               src_i0, dst_i0, src_i1, dst_i1, dsc_i0, dsc_i1,
               S0, T0, S1, T1, EX0, EX1, attv, Z, Zd,
               out_acc, den_acc, gsem0, gsem1, isem0, isem1, ssem0, ssem1):
    cid = lax.axis_index("c")
    sid = lax.axis_index("s")
    wid = sid * NC + cid
    src_i = (src_i0, src_i1)
    dst_i = (dst_i0, dst_i1)
    dsc_i = (dsc_i0, dsc_i1)
    S = (S0, S1)
    T = (T0, T1)
    EX = (EX0, EX1)
    gsem = (gsem0, gsem1)
    isem = (isem0, isem1)
    ssem = (ssem0, ssem1)

    zv = jnp.zeros((L,), jnp.float32)

    def _zrow(i, _):
        for j in range(8):
            Z[i, pl.ds(16 * j, 16)] = zv
        return 0
    lax.fori_loop(0, RCH, _zrow, 0)

    def _zd(i, _):
        Zd[pl.ds(i * 16, 16)] = zv
        return 0
    lax.fori_loop(0, DCH // 16, _zd, 0)

    def _zacc(k, _):
        ch = sid + 16 * k

        @pl.when(ch < NCH)
        def _():
            pltpu.sync_copy(Z, out_acc.at[pl.ds(ch * RCH, RCH)])
        return 0

    lax.fori_loop(0, (NCH + 15) // 16, _zacc, 0)

    for k in range(2):
        ch = sid + 16 * k

        @pl.when(ch < NDC)
        def _():
            pltpu.sync_copy(Zd, den_acc.at[pl.ds(ch * DCH, DCH)])

        @pl.when(ch == NDC)
        def _():
            pltpu.sync_copy(Zd.at[pl.ds(0, N - NDC * DCH)],
                            den_acc.at[pl.ds(NDC * DCH, N - NDC * DCH)])

    plsc.subcore_barrier()

    pltpu.sync_copy(att_hbm, attv)
    att_regs = [attv[pl.ds(16 * j, 16)] for j in range(8)]
    rows16 = jnp.arange(16, dtype=jnp.int32)
    lane0 = rows16 == 0
    ebase = wid * EP

    def _fetch_idx(c, b):
        pltpu.async_copy(src_hbm.at[pl.ds(ebase + c * K, K)], src_i[b], isem[b])
        pltpu.async_copy(dst_hbm.at[pl.ds(ebase + c * K, K)], dst_i[b], isem[b])

    def _wait_idx(b):
        pltpu.make_async_copy(src_hbm.at[pl.ds(ebase, K)], src_i[b],
                              isem[b]).wait()
        pltpu.make_async_copy(dst_hbm.at[pl.ds(ebase, K)], dst_i[b],
                              isem[b]).wait()

    def _gather(b):
        pltpu.async_copy(xl_hbm.at[src_i[b]], S[b], gsem[b])
        pltpu.async_copy(xr_hbm.at[dst_i[b]], T[b], gsem[b])

    def _wait_gather(b):
        pltpu.make_async_copy(xl_hbm.at[src_i[b]], S[b], gsem[b]).wait()
        pltpu.make_async_copy(xr_hbm.at[dst_i[b]], T[b], gsem[b]).wait()

    def _scatter(b):
        pltpu.async_copy(S[b], out_acc.at[dsc_i[b]], ssem[b], add=True)
        pltpu.async_copy(EX[b], den_acc.at[dsc_i[b]], ssem[b], add=True)

    def _wait_scatter(b):
        pltpu.make_async_copy(S[b], out_acc.at[dsc_i[b]], ssem[b]).wait()
        pltpu.make_async_copy(EX[b], den_acc.at[dsc_i[b]], ssem[b]).wait()

    def _compute(b):
        Sb, Tb = S[b], T[b]

        def _cpi(g, _):
            dsc_i[b][pl.ds(g * L, L)] = dst_i[b][pl.ds(g * L, L)]
            return 0

        lax.fori_loop(0, NG, _cpi, 0)

        @plsc.parallel_loop(0, K)
        def _edge(ee):
            eidx = jnp.full((L,), ee, jnp.int32)
            srcs = plsc.load_gather(src_i[b], [eidx])
            dsts = plsc.load_gather(dst_i[b], [eidx])
            srow = [Sb[ee, pl.ds(16 * j, 16)] for j in range(8)]
            acc = zv
            for j in range(8):
                z = srow[j] + Tb[ee, pl.ds(16 * j, 16)]
                acc = acc + jnp.maximum(z, 0.2 * z) * att_regs[j]
            al = jnp.sum(acc)
            exs = jnp.where(srcs != dsts, jnp.exp(jnp.full((L,), al)), 0.0)
            for j in range(8):
                Sb[ee, pl.ds(16 * j, 16)] = srow[j] * exs
            plsc.store_scatter(EX[b], [eidx], exs, mask=lane0)

    pltpu.sync_copy(src_hbm.at[pl.ds(ebase, K)], src_i[0])
    pltpu.sync_copy(dst_hbm.at[pl.ds(ebase, K)], dst_i[0])
    _gather(0)
    _fetch_idx(1, 1)

    def _pair(i, _):
        for b in range(2):
            c = 2 * i + b

            @pl.when(c + 1 < NCHUNK)
            def _():
                _wait_idx(b ^ 1)

                @pl.when(c >= 1)
                def _():
                    _wait_scatter(b ^ 1)

                _gather(b ^ 1)

            _wait_gather(b)
            _compute(b)

            @pl.when(c + 2 < NCHUNK)
            def _():
                _fetch_idx(c + 2, b)

            _scatter(b)
        return 0

    lax.fori_loop(0, NCHUNK // 2, _pair, 0)
    _wait_gather(0)
    _compute(0)
    _wait_scatter(1)
    _scatter(0)
    _wait_scatter(0)

    plsc.subcore_barrier()

    def _dracc(k, _):
        ch = sid + 16 * k

        @pl.when(ch < NCH)
        def _():
            pltpu.sync_copy(out_acc.at[pl.ds(ch * RCH, RCH)], Z)
            pltpu.sync_copy(Z, out_hbm.at[cid, pl.ds(ch * RCH, RCH)])
        return 0

    lax.fori_loop(0, (NCH + 15) // 16, _dracc, 0)

    def _dr_den(den_dst):
        for k in range(2):
            ch = sid + 16 * k

            @pl.when(ch < NDC)
            def _():
                pltpu.sync_copy(den_acc.at[pl.ds(ch * DCH, DCH)], Zd)
                pltpu.sync_copy(Zd, den_dst.at[pl.ds(ch * DCH, DCH)])

            @pl.when(ch == NDC)
            def _():
                nrem = N - NDC * DCH
                pltpu.sync_copy(den_acc.at[pl.ds(NDC * DCH, nrem)],
                                Zd.at[pl.ds(0, nrem)])
                pltpu.sync_copy(Zd.at[pl.ds(0, nrem)],
                                den_dst.at[pl.ds(NDC * DCH, nrem)])

    @pl.when(cid == 0)
    def _():
        _dr_den(den0_hbm)

    @pl.when(cid == 1)
    def _():
        _dr_den(den1_hbm)


@functools.partial(jax.jit, static_argnums=())
def _edges(xl, xr, src, dst, att):
    f = pl.kernel(
        _edge_body,
        mesh=plsc.VectorSubcoreMesh(core_axis_name="c", subcore_axis_name="s",
                                    num_cores=NC),
        compiler_params=pltpu.CompilerParams(needs_layout_passes=False),
        out_type=[
            jax.ShapeDtypeStruct((NC, N, D), jnp.float32),
            jax.ShapeDtypeStruct((N,), jnp.float32),
            jax.ShapeDtypeStruct((N,), jnp.float32),
        ],
        scratch_types=[
            pltpu.VMEM((K,), jnp.int32),
            pltpu.VMEM((K,), jnp.int32),
            pltpu.VMEM((K,), jnp.int32),
            pltpu.VMEM((K,), jnp.int32),
            pltpu.VMEM((K,), jnp.int32),
            pltpu.VMEM((K,), jnp.int32),
            pltpu.VMEM((K, D), jnp.float32),
            pltpu.VMEM((K, D), jnp.float32),
            pltpu.VMEM((K, D), jnp.float32),
            pltpu.VMEM((K, D), jnp.float32),
            pltpu.VMEM((K,), jnp.float32),
            pltpu.VMEM((K,), jnp.float32),
            pltpu.VMEM((D,), jnp.float32),
            pltpu.VMEM((RCH, D), jnp.float32),
            pltpu.VMEM((DCH,), jnp.float32),
            pltpu.VMEM_SHARED((N, D), jnp.float32),
            pltpu.VMEM_SHARED((N,), jnp.float32),
            pltpu.SemaphoreType.DMA,
            pltpu.SemaphoreType.DMA,
            pltpu.SemaphoreType.DMA,
            pltpu.SemaphoreType.DMA,
            pltpu.SemaphoreType.DMA,
            pltpu.SemaphoreType.DMA,
        ],
    )
    return f(xl, xr, att.reshape(D), src, dst)


def kernel(x, edge_index, W_l0, b_l0, W_r0, b_r0, att0, bias0, W_res0, g0, be0,
           W_l1, b_l1, W_r1, b_r1, att1, bias1, W_res1, g1, be1):
    src = edge_index[0]
    dst = edge_index[1]
    xl0, xr0, res0 = _project(x, W_l0, b_l0, W_r0, b_r0, W_res0)
    o, d0, d1 = _edges(xl0, xr0, src, dst, att0)
    xl1, xr1, res1 = _mid(xl0, xr0, res0, o, d0.reshape(N, 1),
                          d1.reshape(N, 1), att0, bias0, g0, be0,
                          W_l1, b_l1, W_r1, b_r1, W_res1)
    o, d0, d1 = _edges(xl1, xr1, src, dst, att1)
    return _post(xl1, xr1, res1, o, d0.reshape(N, 1), d1.reshape(N, 1),
                 att1, bias1, g1, be1)
